# Initial kernel scaffold; baseline (speedup 1.0000x reference)
import jax
import jax.numpy as jnp
from jax.experimental import pallas as pl


def kernel(input_ids, beta):
    raise NotImplementedError("write your pallas kernel here")


# speedup vs baseline: 2.3602x; 2.3602x over previous
import functools

import jax
import jax.numpy as jnp
from jax import lax
from jax.experimental import pallas as pl
from jax.experimental.pallas import tpu as pltpu
from jax.experimental.pallas import tpu_sc as plsc

_VOCAB = 100000
_B = 1024
_L = 200
_LANES = 16
_NFULL = 12
_TAIL_OFF = _L - _LANES
_NVEC = _NFULL + 1

_info = plsc.get_sparse_core_info()
_NW = _info.num_cores * _info.num_subcores
_ROWS_PER_W = _B // _NW


def _row_offsets():
    return [k * _LANES for k in range(_NFULL)] + [_TAIL_OFF]


def _sc_body(ids_hbm, beta_hbm, zeros_hbm, out_hbm, ids_v, rowbuf, beta_v):
    wid = lax.axis_index("s") * _info.num_cores + lax.axis_index("c")
    base = wid * _ROWS_PER_W

    pltpu.sync_copy(ids_hbm.at[pl.ds(base, _ROWS_PER_W)], ids_v)
    pltpu.sync_copy(zeros_hbm, rowbuf)
    pltpu.sync_copy(beta_hbm, beta_v)
    ebeta = jnp.exp(beta_v[...])

    lane = lax.broadcasted_iota(jnp.int32, (_LANES,), 0)
    tail_mask = lane >= (_LANES - (_L - _NFULL * _LANES))
    ones = jnp.full((_LANES,), 1.0, dtype=jnp.float32)
    zval = jnp.zeros((_LANES,), dtype=jnp.float32)
    offsets = _row_offsets()

    def per_row(r, carry):
        idxs = [ids_v[r, pl.ds(off, _LANES)] for off in offsets]
        for k, idx in enumerate(idxs):
            mask = tail_mask if k == _NFULL else None
            plsc.addupdate_scatter(rowbuf, [idx], ones, mask=mask)
        counts = [plsc.load_gather(rowbuf, [idx]) for idx in idxs]
        for idx, c in zip(idxs, counts):
            v = c / (c + ebeta)
            v = jnp.where(idx == 1, 0.0, v)
            plsc.store_scatter(rowbuf, [idx], v)
        pltpu.sync_copy(rowbuf, out_hbm.at[base + r])
        for idx in idxs:
            plsc.store_scatter(rowbuf, [idx], zval)
        return carry

    lax.fori_loop(0, _ROWS_PER_W, per_row, 0)


@jax.jit
def _encode(input_ids, beta_v, zeros_row):
    mesh = plsc.VectorSubcoreMesh(core_axis_name="c", subcore_axis_name="s")
    return pl.kernel(
        _sc_body,
        out_type=jax.ShapeDtypeStruct((_B, _VOCAB), jnp.float32),
        mesh=mesh,
        scratch_types=[
            pltpu.VMEM((_ROWS_PER_W, _L), jnp.int32),
            pltpu.VMEM((_VOCAB,), jnp.float32),
            pltpu.VMEM((_LANES,), jnp.float32),
        ],
        compiler_params=pltpu.CompilerParams(needs_layout_passes=False),
    )(input_ids, beta_v, zeros_row)


def kernel(input_ids, beta):
    beta_v = jnp.full((_LANES,), beta, dtype=jnp.float32)
    zeros_row = jnp.zeros((_VOCAB,), dtype=jnp.float32)
    return _encode(input_ids, beta_v, zeros_row)
